# baseline (speedup 1.0000x reference)
import functools

import jax
import jax.numpy as jnp
from jax import lax
from jax.experimental import pallas as pl
from jax.experimental.pallas import tpu as pltpu
from jax.experimental.pallas import tpu_sc as plsc

_NUM_CORES = 2
_NUM_SUBCORES = 16
_NUM_WORKERS = _NUM_CORES * _NUM_SUBCORES
_LANES = 16
_CHUNK = 256


def _mf_forward(user_idx, item_idx, W, H):
    B = user_idx.shape[0]
    K = W.shape[1]
    b_per_w = B // _NUM_WORKERS
    n_chunks = b_per_w // _CHUNK
    groups = _CHUNK // _LANES

    mesh = plsc.VectorSubcoreMesh(core_axis_name="c", subcore_axis_name="s")

    @functools.partial(
        pl.kernel,
        mesh=mesh,
        out_type=jax.ShapeDtypeStruct((B,), jnp.float32),
        scratch_types=[
            pltpu.VMEM((b_per_w,), jnp.int32),
            pltpu.VMEM((b_per_w,), jnp.int32),
            pltpu.VMEM((_CHUNK, K), jnp.float32),
            pltpu.VMEM((_CHUNK, K), jnp.float32),
            pltpu.VMEM((b_per_w,), jnp.float32),
            pltpu.SemaphoreType.DMA((8,)),
            pltpu.SemaphoreType.DMA,
        ],
        compiler_params=pltpu.CompilerParams(needs_layout_passes=False),
    )
    def mf_kernel(uidx_hbm, vidx_hbm, w_hbm, h_hbm, out_hbm,
                  uidx_v, vidx_v, u_rows, v_rows, out_v, sems, sem):
        wid = lax.axis_index("s") * _NUM_CORES + lax.axis_index("c")
        base = wid * b_per_w

        pltpu.sync_copy(uidx_hbm.at[pl.ds(base, b_per_w)], uidx_v)
        pltpu.sync_copy(vidx_hbm.at[pl.ds(base, b_per_w)], vidx_v)

        lane = lax.iota(jnp.int32, _LANES)

        def chunk_body(c, carry):
            cbase = c * _CHUNK

            def fetch_body(g, inner):
                u16 = uidx_v[pl.ds(cbase + g * _LANES, _LANES)]
                v16 = vidx_v[pl.ds(cbase + g * _LANES, _LANES)]
                for j in range(_LANES):
                    dst = g * _LANES + j
                    pltpu.async_copy(w_hbm.at[u16[j]], u_rows.at[dst],
                                     sems.at[j % 8])
                    pltpu.async_copy(h_hbm.at[v16[j]], v_rows.at[dst],
                                     sems.at[j % 8])
                return inner

            lax.fori_loop(0, _CHUNK // _LANES, fetch_body, 0)
            for j in range(8):
                pltpu.make_async_copy(
                    w_hbm.at[pl.ds(0, _CHUNK // 8)],
                    u_rows.at[pl.ds(j * (_CHUNK // 8), _CHUNK // 8)],
                    sems.at[j]).wait()
                pltpu.make_async_copy(
                    w_hbm.at[pl.ds(0, _CHUNK // 8)],
                    v_rows.at[pl.ds(j * (_CHUNK // 8), _CHUNK // 8)],
                    sems.at[j]).wait()

            def group_body(g, inner):
                rows = g * _LANES + lane
                acc = jnp.zeros((_LANES,), jnp.float32)
                for kk in range(K):
                    col = (lane + kk) & (K - 1)
                    u = plsc.load_gather(u_rows, [rows, col])
                    v = plsc.load_gather(v_rows, [rows, col])
                    acc = acc + u * v
                out_v[pl.ds(cbase + g * _LANES, _LANES)] = (
                    1.0 / (1.0 + jnp.exp(-acc)))
                return inner

            lax.fori_loop(0, groups, group_body, 0)
            return carry

        lax.fori_loop(0, n_chunks, chunk_body, 0)
        pltpu.sync_copy(out_v, out_hbm.at[pl.ds(base, b_per_w)])

    return mf_kernel(user_idx, item_idx, W, H)


def kernel(x, W, H):
    user_idx = x[:, 0].astype(jnp.int32)
    item_idx = x[:, 1].astype(jnp.int32)
    return _mf_forward(user_idx, item_idx, W, H)


# speedup vs baseline: 1.0048x; 1.0048x over previous
import functools

import jax
import jax.numpy as jnp
from jax import lax
from jax.experimental import pallas as pl
from jax.experimental.pallas import tpu as pltpu
from jax.experimental.pallas import tpu_sc as plsc

_NUM_CORES = 2
_NUM_SUBCORES = 16
_NUM_WORKERS = _NUM_CORES * _NUM_SUBCORES
_LANES = 16
_CHUNK = 256
_PACK = 4


def _mf_forward(user_idx, item_idx, W, H):
    B = user_idx.shape[0]
    K = W.shape[1]
    b_per_w = B // _NUM_WORKERS
    n_chunks = b_per_w // _CHUNK
    groups = _CHUNK // _LANES

    mesh = plsc.VectorSubcoreMesh(core_axis_name="c", subcore_axis_name="s")

    @functools.partial(
        pl.kernel,
        mesh=mesh,
        out_type=jax.ShapeDtypeStruct((B,), jnp.float32),
        scratch_types=[
            pltpu.VMEM((b_per_w,), jnp.int32),
            pltpu.VMEM((b_per_w,), jnp.int32),
            pltpu.VMEM((_CHUNK // _PACK, _PACK * K), jnp.float32),
            pltpu.VMEM((_CHUNK // _PACK, _PACK * K), jnp.float32),
            pltpu.VMEM((b_per_w,), jnp.float32),
            pltpu.SemaphoreType.DMA,
        ],
        compiler_params=pltpu.CompilerParams(needs_layout_passes=False),
    )
    def mf_kernel(uidx_hbm, vidx_hbm, w_hbm, h_hbm, out_hbm,
                  uidx_v, vidx_v, u_rows, v_rows, out_v, sem):
        wid = lax.axis_index("s") * _NUM_CORES + lax.axis_index("c")
        base = wid * b_per_w

        pltpu.sync_copy(uidx_hbm.at[pl.ds(base, b_per_w)], uidx_v)
        pltpu.sync_copy(vidx_hbm.at[pl.ds(base, b_per_w)], vidx_v)

        lane = lax.iota(jnp.int32, _LANES)

        def chunk_body(c, carry):
            cbase = c * _CHUNK

            def fetch_body(g, inner):
                u16 = uidx_v[pl.ds(cbase + g * _LANES, _LANES)]
                v16 = vidx_v[pl.ds(cbase + g * _LANES, _LANES)]
                for j in range(_LANES):
                    r = g * _LANES + j
                    dst_row = r // _PACK
                    dst_col = pl.ds((r % _PACK) * K, K)
                    pltpu.async_copy(w_hbm.at[u16[j]],
                                     u_rows.at[dst_row, dst_col], sem)
                    pltpu.async_copy(h_hbm.at[v16[j]],
                                     v_rows.at[dst_row, dst_col], sem)
                return inner

            lax.fori_loop(0, _CHUNK // _LANES, fetch_body, 0)
            n_waits = (2 * _CHUNK * K) // b_per_w
            for _ in range(n_waits):
                pltpu.make_async_copy(out_hbm.at[pl.ds(0, b_per_w)],
                                      out_v, sem).wait()

            def group_body(g, inner):
                rows = g * _LANES + lane
                prow = rows // _PACK
                pbase = (rows % _PACK) * K
                acc = jnp.zeros((_LANES,), jnp.float32)
                for kk in range(K):
                    col = pbase + ((lane + kk) & (K - 1))
                    u = plsc.load_gather(u_rows, [prow, col])
                    v = plsc.load_gather(v_rows, [prow, col])
                    acc = acc + u * v
                out_v[pl.ds(cbase + g * _LANES, _LANES)] = (
                    1.0 / (1.0 + jnp.exp(-acc)))
                return inner

            lax.fori_loop(0, groups, group_body, 0)
            return carry

        lax.fori_loop(0, n_chunks, chunk_body, 0)
        pltpu.sync_copy(out_v, out_hbm.at[pl.ds(base, b_per_w)])

    return mf_kernel(user_idx, item_idx, W, H)


def kernel(x, W, H):
    user_idx = x[:, 0].astype(jnp.int32)
    item_idx = x[:, 1].astype(jnp.int32)
    return _mf_forward(user_idx, item_idx, W, H)


# speedup vs baseline: 1.0065x; 1.0018x over previous
import functools

import jax
import jax.numpy as jnp
from jax import lax
from jax.experimental import pallas as pl
from jax.experimental.pallas import tpu as pltpu
from jax.experimental.pallas import tpu_sc as plsc

_NUM_CORES = 2
_NUM_SUBCORES = 16
_NUM_WORKERS = _NUM_CORES * _NUM_SUBCORES
_LANES = 16
_CHUNK = 256
_PACK = 4


def _mf_forward(user_idx, item_idx, W, H):
    B = user_idx.shape[0]
    K = W.shape[1]
    b_per_w = B // _NUM_WORKERS
    n_chunks = b_per_w // _CHUNK
    groups = _CHUNK // _LANES

    mesh = plsc.VectorSubcoreMesh(core_axis_name="c", subcore_axis_name="s")

    @functools.partial(
        pl.kernel,
        mesh=mesh,
        out_type=jax.ShapeDtypeStruct((B,), jnp.float32),
        scratch_types=[
            pltpu.VMEM((b_per_w,), jnp.int32),
            pltpu.VMEM((b_per_w,), jnp.int32),
            pltpu.VMEM((_CHUNK // _PACK, _PACK * K), jnp.float32),
            pltpu.VMEM((_CHUNK // _PACK, _PACK * K), jnp.float32),
            pltpu.VMEM((_CHUNK // _PACK, _PACK * K), jnp.float32),
            pltpu.VMEM((_CHUNK // _PACK, _PACK * K), jnp.float32),
            pltpu.VMEM((b_per_w,), jnp.float32),
            pltpu.SemaphoreType.DMA,
            pltpu.SemaphoreType.DMA,
        ],
        compiler_params=pltpu.CompilerParams(needs_layout_passes=False),
    )
    def mf_kernel(uidx_hbm, vidx_hbm, w_hbm, h_hbm, out_hbm,
                  uidx_v, vidx_v, u_a, v_a, u_b, v_b, out_v, sem_a, sem_b):
        wid = lax.axis_index("s") * _NUM_CORES + lax.axis_index("c")
        base = wid * b_per_w

        pltpu.sync_copy(uidx_hbm.at[pl.ds(base, b_per_w)], uidx_v)
        pltpu.sync_copy(vidx_hbm.at[pl.ds(base, b_per_w)], vidx_v)

        lane = lax.iota(jnp.int32, _LANES)

        def fire(cbase, u_rows, v_rows, sem):
            def fetch_body(g, inner):
                u16 = uidx_v[pl.ds(cbase + g * _LANES, _LANES)]
                v16 = vidx_v[pl.ds(cbase + g * _LANES, _LANES)]
                for j in range(_LANES):
                    r = g * _LANES + j
                    dst_row = r // _PACK
                    dst_col = pl.ds((r % _PACK) * K, K)
                    pltpu.async_copy(w_hbm.at[u16[j]],
                                     u_rows.at[dst_row, dst_col], sem)
                    pltpu.async_copy(h_hbm.at[v16[j]],
                                     v_rows.at[dst_row, dst_col], sem)
                return inner

            lax.fori_loop(0, _CHUNK // _LANES, fetch_body, 0)

        def drain(sem):
            n_waits = (2 * _CHUNK * K) // b_per_w
            for _ in range(n_waits):
                pltpu.make_async_copy(out_hbm.at[pl.ds(0, b_per_w)],
                                      out_v, sem).wait()

        def compute(cbase, u_rows, v_rows):
            def group_body(g, inner):
                rows = g * _LANES + lane
                prow = rows // _PACK
                pbase = (rows % _PACK) * K
                acc = jnp.zeros((_LANES,), jnp.float32)
                for kk in range(K):
                    col = pbase + ((lane + kk) & (K - 1))
                    u = plsc.load_gather(u_rows, [prow, col])
                    v = plsc.load_gather(v_rows, [prow, col])
                    acc = acc + u * v
                out_v[pl.ds(cbase + g * _LANES, _LANES)] = (
                    1.0 / (1.0 + jnp.exp(-acc)))
                return inner

            lax.fori_loop(0, groups, group_body, 0)

        fire(0, u_a, v_a, sem_a)
        fire(_CHUNK, u_b, v_b, sem_b)
        drain(sem_a)
        compute(0, u_a, v_a)
        drain(sem_b)
        compute(_CHUNK, u_b, v_b)
        pltpu.sync_copy(out_v, out_hbm.at[pl.ds(base, b_per_w)])

    return mf_kernel(user_idx, item_idx, W, H)


def kernel(x, W, H):
    user_idx = x[:, 0].astype(jnp.int32)
    item_idx = x[:, 1].astype(jnp.int32)
    return _mf_forward(user_idx, item_idx, W, H)
